# baseline (speedup 1.0000x reference)
import jax
import jax.numpy as jnp
from jax import lax
from jax.experimental import pallas as pl
from jax.experimental.pallas import tpu as pltpu
from jax.experimental.pallas import tpu_sc as plsc

N = 10000
D = 128
E = 320000

NP = 10240
SL = 1792
NSL = 6
RPT = SL // 16
C = 128
NCHUNK = 160
PER_W = C * NCHUNK
EPAD = 16 * PER_W
EROWS = EPAD // 128
BN = 1024


def _prep_body(src_r, dst_r, hop_r, gidx1_r, sidx0_r, sidx1_r):
    src = src_r[...]
    dst = dst_r[...]
    hop = hop_r[...]
    gidx1_r[...] = src + jnp.where(hop == 2, NP, 0)
    act0 = hop == 1
    act1 = hop >= 1
    for k in range(NSL):
        loc = dst - k * SL
        in_k = (dst >= k * SL) & (dst < (k + 1) * SL)
        sidx0_r[k, ...] = jnp.where(act0 & in_k, loc, SL)
        sidx1_r[k, ...] = jnp.where(act1 & in_k, loc, SL)


def _mm2_body(x_r, w10_r, b10_r, w21_r, b21_r, h0_r, h2_r):
    xb = x_r[...]
    h0_r[...] = jnp.dot(xb, w10_r[...], preferred_element_type=jnp.float32) + b10_r[...]
    h2_r[...] = jnp.dot(xb, w21_r[...], preferred_element_type=jnp.float32) + b21_r[...]


def _mid_body(x_r, a_r, w11_r, b11_r, x1_r, h1_r):
    cur = x_r[...] + jnp.maximum(a_r[...], 0.0)
    nrm = jnp.maximum(jnp.sqrt(jnp.sum(cur * cur, axis=1, keepdims=True)), 1e-12)
    x1 = cur / nrm
    x1_r[...] = x1
    h1_r[...] = jnp.dot(x1, w11_r[...], preferred_element_type=jnp.float32) + b11_r[...]


def _fin_body(x1_r, a_r, out_r):
    cur = x1_r[...] + jnp.maximum(a_r[...], 0.0)
    nrm = jnp.maximum(jnp.sqrt(jnp.sum(cur * cur, axis=1, keepdims=True)), 1e-12)
    out_r[...] = cur / nrm


def _sc_sweep(tab, gidx, sidx, zeros, out,
              g2, s2, r0b, r1b, wb, acc,
              gs0, gs1, ss0, ss1):
    c = lax.axis_index("c")
    s = lax.axis_index("s")
    rows = (r0b, r1b)
    gsem = (gs0, gs1)
    ssem = (ss0, ss1)
    w0 = pl.multiple_of(s * NCHUNK, 8)
    r0 = pl.multiple_of(s * RPT, 8)
    pltpu.sync_copy(gidx.at[pl.ds(w0, NCHUNK)], g2)

    def fire_gather(j, b):
        pltpu.async_copy(tab.at[g2.at[j]], rows[b], gsem[b])

    def wait_gather(j, b):
        pltpu.make_async_copy(tab.at[g2.at[j]], rows[b], gsem[b]).wait()

    def fire_scatter(j, b):
        pltpu.async_copy(rows[b], acc.at[s2.at[j]], ssem[b], add=True)

    def wait_scatter(j, b):
        pltpu.make_async_copy(rows[b], acc.at[s2.at[j]], ssem[b]).wait()

    def one_pass(p, carry):
        sl = 2 * p + c
        pltpu.sync_copy(
            sidx.at[pl.ds(pl.multiple_of(sl * EROWS + w0, 8), NCHUNK)], s2)
        pltpu.sync_copy(zeros, wb)
        pltpu.sync_copy(wb, acc.at[pl.ds(r0, RPT)])
        plsc.subcore_barrier()

        fire_gather(0, 0)
        wait_gather(0, 0)
        fire_scatter(0, 0)
        fire_gather(1, 1)

        def duo(i, carry):
            for u in range(2):
                j = 1 + 2 * i + u
                b = (1 + u) % 2
                wait_gather(j, b)
                fire_scatter(j, b)
                wait_scatter(j - 1, 1 - b)
                fire_gather(j + 1, 1 - b)
            return carry

        lax.fori_loop(0, (NCHUNK - 2) // 2, duo, 0)
        j = NCHUNK - 1
        wait_gather(j, j % 2)
        fire_scatter(j, j % 2)
        wait_scatter(j - 1, (j - 1) % 2)
        wait_scatter(j, j % 2)

        plsc.subcore_barrier()
        pltpu.sync_copy(acc.at[pl.ds(r0, RPT)], wb)
        pltpu.sync_copy(
            wb, out.at[pl.ds(pl.multiple_of(sl * SL + r0, 8), RPT)])
        return carry

    lax.fori_loop(0, 3, one_pass, 0)


def _make_sc_sweep():
    mesh = plsc.VectorSubcoreMesh(core_axis_name="c", subcore_axis_name="s")
    return pl.kernel(
        _sc_sweep,
        out_type=jax.ShapeDtypeStruct((NSL * SL, D), jnp.float32),
        mesh=mesh,
        name="sc_edge_sweep",
        scratch_types=[
            pltpu.VMEM((NCHUNK, C), jnp.int32),
            pltpu.VMEM((NCHUNK, C), jnp.int32),
            pltpu.VMEM((C, D), jnp.float32),
            pltpu.VMEM((C, D), jnp.float32),
            pltpu.VMEM((RPT, D), jnp.float32),
            pltpu.VMEM_SHARED((SL + 8, D), jnp.float32),
            pltpu.SemaphoreType.DMA,
            pltpu.SemaphoreType.DMA,
            pltpu.SemaphoreType.DMA,
            pltpu.SemaphoreType.DMA,
        ],
    )


def kernel(x, edge_index, edge_attr, W_k1_t0, b_k1_t0, W_k1_t1, b_k1_t1,
           W_k2_t1, b_k2_t1):
    f32 = jnp.float32

    xp = jnp.zeros((NP, D), f32).at[:N].set(x)
    src = jnp.zeros((EPAD,), jnp.int32).at[:E].set(edge_index[0]).reshape(EROWS, 128)
    dst = jnp.zeros((EPAD,), jnp.int32).at[:E].set(edge_index[1]).reshape(EROWS, 128)
    hop = jnp.zeros((EPAD,), jnp.int32).at[:E].set(edge_attr[:, 0]).reshape(EROWS, 128)
    b10 = b_k1_t0.reshape(1, D)
    b11 = b_k1_t1.reshape(1, D)
    b21 = b_k2_t1.reshape(1, D)
    zeros_wb = jnp.zeros((RPT, D), f32)

    gidx1, sidx0, sidx1 = pl.pallas_call(
        _prep_body,
        in_specs=[pl.BlockSpec((EROWS, 128), lambda: (0, 0))] * 3,
        out_specs=[
            pl.BlockSpec((EROWS, 128), lambda: (0, 0)),
            pl.BlockSpec((NSL, EROWS, 128), lambda: (0, 0, 0)),
            pl.BlockSpec((NSL, EROWS, 128), lambda: (0, 0, 0)),
        ],
        out_shape=[
            jax.ShapeDtypeStruct((EROWS, 128), jnp.int32),
            jax.ShapeDtypeStruct((NSL, EROWS, 128), jnp.int32),
            jax.ShapeDtypeStruct((NSL, EROWS, 128), jnp.int32),
        ],
    )(src, dst, hop)

    nb = NP // BN
    full = pl.BlockSpec((D, D), lambda i: (0, 0))
    brow = pl.BlockSpec((1, D), lambda i: (0, 0))
    blk = pl.BlockSpec((BN, D), lambda i: (i, 0))
    h0, h2 = pl.pallas_call(
        _mm2_body,
        grid=(nb,),
        in_specs=[blk, full, brow, full, brow],
        out_specs=[blk, blk],
        out_shape=[jax.ShapeDtypeStruct((NP, D), f32)] * 2,
    )(xp, W_k1_t0, b10, W_k2_t1, b21)

    sweep = _make_sc_sweep()

    tab0 = jnp.concatenate([h0, h2], axis=0)
    acc0 = sweep(tab0, src, sidx0.reshape(NSL * EROWS, 128), zeros_wb)

    x1, h1 = pl.pallas_call(
        _mid_body,
        grid=(nb,),
        in_specs=[blk, blk, full, brow],
        out_specs=[blk, blk],
        out_shape=[jax.ShapeDtypeStruct((NP, D), f32)] * 2,
    )(xp, acc0, W_k1_t1, b11)

    tab1 = jnp.concatenate([h1, h2], axis=0)
    acc1 = sweep(tab1, gidx1, sidx1.reshape(NSL * EROWS, 128), zeros_wb)

    out = pl.pallas_call(
        _fin_body,
        grid=(nb,),
        in_specs=[blk, blk],
        out_specs=blk,
        out_shape=jax.ShapeDtypeStruct((NP, D), f32),
    )(x1, acc1)

    return out[:N]


# speedup vs baseline: 1.1145x; 1.1145x over previous
import jax
import jax.numpy as jnp
from jax import lax
from jax.experimental import pallas as pl
from jax.experimental.pallas import tpu as pltpu
from jax.experimental.pallas import tpu_sc as plsc

N = 10000
D = 128
E = 320000

NP = 10240
SL = 1792
NSL = 6
RPT = SL // 16
C = 128
NCHUNK = 160
PER_W = C * NCHUNK
EPAD = 16 * PER_W
EROWS = EPAD // 128
BN = 1024


def _prep_body(src_r, dst_r, hop_r, gidx1_r, sidx0_r, sidx1_r):
    src = src_r[...]
    dst = dst_r[...]
    hop = hop_r[...]
    gidx1_r[...] = src + jnp.where(hop == 2, NP, 0)
    act0 = hop == 1
    act1 = hop >= 1
    trash = SL + lax.broadcasted_iota(jnp.int32, (EROWS, 128), 1)
    for k in range(NSL):
        loc = dst - k * SL
        in_k = (dst >= k * SL) & (dst < (k + 1) * SL)
        sidx0_r[k, ...] = jnp.where(act0 & in_k, loc, trash)
        sidx1_r[k, ...] = jnp.where(act1 & in_k, loc, trash)


def _mm2_body(x_r, w10_r, b10_r, w21_r, b21_r, h0_r, h2_r):
    xb = x_r[...]
    h0_r[...] = jnp.dot(xb, w10_r[...], preferred_element_type=jnp.float32) + b10_r[...]
    h2_r[...] = jnp.dot(xb, w21_r[...], preferred_element_type=jnp.float32) + b21_r[...]


def _mid_body(x_r, a_r, w11_r, b11_r, x1_r, h1_r):
    cur = x_r[...] + jnp.maximum(a_r[...], 0.0)
    nrm = jnp.maximum(jnp.sqrt(jnp.sum(cur * cur, axis=1, keepdims=True)), 1e-12)
    x1 = cur / nrm
    x1_r[...] = x1
    h1_r[...] = jnp.dot(x1, w11_r[...], preferred_element_type=jnp.float32) + b11_r[...]


def _fin_body(x1_r, a_r, out_r):
    cur = x1_r[...] + jnp.maximum(a_r[...], 0.0)
    nrm = jnp.maximum(jnp.sqrt(jnp.sum(cur * cur, axis=1, keepdims=True)), 1e-12)
    out_r[...] = cur / nrm


def _sc_sweep(tab, gidx, sidx, zeros, out,
              g2, s2, r0b, r1b, wb, acc,
              gs0, gs1, ss0, ss1):
    c = lax.axis_index("c")
    s = lax.axis_index("s")
    rows = (r0b, r1b)
    gsem = (gs0, gs1)
    ssem = (ss0, ss1)
    w0 = pl.multiple_of(s * NCHUNK, 8)
    r0 = pl.multiple_of(s * RPT, 8)
    pltpu.sync_copy(gidx.at[pl.ds(w0, NCHUNK)], g2)

    def fire_gather(j, b):
        pltpu.async_copy(tab.at[g2.at[j]], rows[b], gsem[b])

    def wait_gather(j, b):
        pltpu.make_async_copy(tab.at[g2.at[j]], rows[b], gsem[b]).wait()

    def fire_scatter(j, b):
        pltpu.async_copy(rows[b], acc.at[s2.at[j]], ssem[b], add=True)

    def wait_scatter(j, b):
        pltpu.make_async_copy(rows[b], acc.at[s2.at[j]], ssem[b]).wait()

    def one_pass(p, carry):
        sl = 2 * p + c
        pltpu.sync_copy(
            sidx.at[pl.ds(pl.multiple_of(sl * EROWS + w0, 8), NCHUNK)], s2)
        pltpu.sync_copy(zeros, wb)
        pltpu.sync_copy(wb, acc.at[pl.ds(r0, RPT)])
        plsc.subcore_barrier()

        fire_gather(0, 0)
        wait_gather(0, 0)
        fire_scatter(0, 0)
        fire_gather(1, 1)

        def duo(i, carry):
            for u in range(2):
                j = 1 + 2 * i + u
                b = (1 + u) % 2
                wait_gather(j, b)
                fire_scatter(j, b)
                wait_scatter(j - 1, 1 - b)
                fire_gather(j + 1, 1 - b)
            return carry

        lax.fori_loop(0, (NCHUNK - 2) // 2, duo, 0)
        j = NCHUNK - 1
        wait_gather(j, j % 2)
        fire_scatter(j, j % 2)
        wait_scatter(j - 1, (j - 1) % 2)
        wait_scatter(j, j % 2)

        plsc.subcore_barrier()
        pltpu.sync_copy(acc.at[pl.ds(r0, RPT)], wb)
        pltpu.sync_copy(
            wb, out.at[pl.ds(pl.multiple_of(sl * SL + r0, 8), RPT)])
        return carry

    lax.fori_loop(0, 3, one_pass, 0)


def _make_sc_sweep():
    mesh = plsc.VectorSubcoreMesh(core_axis_name="c", subcore_axis_name="s")
    return pl.kernel(
        _sc_sweep,
        out_type=jax.ShapeDtypeStruct((NSL * SL, D), jnp.float32),
        mesh=mesh,
        name="sc_edge_sweep",
        scratch_types=[
            pltpu.VMEM((NCHUNK, C), jnp.int32),
            pltpu.VMEM((NCHUNK, C), jnp.int32),
            pltpu.VMEM((C, D), jnp.float32),
            pltpu.VMEM((C, D), jnp.float32),
            pltpu.VMEM((RPT, D), jnp.float32),
            pltpu.VMEM_SHARED((SL + 128, D), jnp.float32),
            pltpu.SemaphoreType.DMA,
            pltpu.SemaphoreType.DMA,
            pltpu.SemaphoreType.DMA,
            pltpu.SemaphoreType.DMA,
        ],
    )


def kernel(x, edge_index, edge_attr, W_k1_t0, b_k1_t0, W_k1_t1, b_k1_t1,
           W_k2_t1, b_k2_t1):
    f32 = jnp.float32

    xp = jnp.zeros((NP, D), f32).at[:N].set(x)
    src = jnp.zeros((EPAD,), jnp.int32).at[:E].set(edge_index[0]).reshape(EROWS, 128)
    dst = jnp.zeros((EPAD,), jnp.int32).at[:E].set(edge_index[1]).reshape(EROWS, 128)
    hop = jnp.zeros((EPAD,), jnp.int32).at[:E].set(edge_attr[:, 0]).reshape(EROWS, 128)
    b10 = b_k1_t0.reshape(1, D)
    b11 = b_k1_t1.reshape(1, D)
    b21 = b_k2_t1.reshape(1, D)
    zeros_wb = jnp.zeros((RPT, D), f32)

    gidx1, sidx0, sidx1 = pl.pallas_call(
        _prep_body,
        in_specs=[pl.BlockSpec((EROWS, 128), lambda: (0, 0))] * 3,
        out_specs=[
            pl.BlockSpec((EROWS, 128), lambda: (0, 0)),
            pl.BlockSpec((NSL, EROWS, 128), lambda: (0, 0, 0)),
            pl.BlockSpec((NSL, EROWS, 128), lambda: (0, 0, 0)),
        ],
        out_shape=[
            jax.ShapeDtypeStruct((EROWS, 128), jnp.int32),
            jax.ShapeDtypeStruct((NSL, EROWS, 128), jnp.int32),
            jax.ShapeDtypeStruct((NSL, EROWS, 128), jnp.int32),
        ],
    )(src, dst, hop)

    nb = NP // BN
    full = pl.BlockSpec((D, D), lambda i: (0, 0))
    brow = pl.BlockSpec((1, D), lambda i: (0, 0))
    blk = pl.BlockSpec((BN, D), lambda i: (i, 0))
    h0, h2 = pl.pallas_call(
        _mm2_body,
        grid=(nb,),
        in_specs=[blk, full, brow, full, brow],
        out_specs=[blk, blk],
        out_shape=[jax.ShapeDtypeStruct((NP, D), f32)] * 2,
    )(xp, W_k1_t0, b10, W_k2_t1, b21)

    sweep = _make_sc_sweep()

    tab0 = jnp.concatenate([h0, h2], axis=0)
    acc0 = sweep(tab0, src, sidx0.reshape(NSL * EROWS, 128), zeros_wb)

    x1, h1 = pl.pallas_call(
        _mid_body,
        grid=(nb,),
        in_specs=[blk, blk, full, brow],
        out_specs=[blk, blk],
        out_shape=[jax.ShapeDtypeStruct((NP, D), f32)] * 2,
    )(xp, acc0, W_k1_t1, b11)

    tab1 = jnp.concatenate([h1, h2], axis=0)
    acc1 = sweep(tab1, gidx1, sidx1.reshape(NSL * EROWS, 128), zeros_wb)

    out = pl.pallas_call(
        _fin_body,
        grid=(nb,),
        in_specs=[blk, blk],
        out_specs=blk,
        out_shape=jax.ShapeDtypeStruct((NP, D), f32),
    )(x1, acc1)

    return out[:N]
